# baseline (speedup 1.0000x reference)
import functools

import jax
import jax.numpy as jnp
from jax import lax
from jax.experimental import pallas as pl
from jax.experimental.pallas import tpu as pltpu
from jax.experimental.pallas import tpu_sc as plsc

EMB = 1024
HEADS = 8
DH = 128
T = 4096
NHASH = 4
NBUCKETS = 64
NCHUNKS = NHASH * NBUCKETS
CS = 64
ROWB = 256
NROWB = T // ROWB


def _layernorm(x, g, b):
    m = jnp.mean(x, axis=-1, keepdims=True)
    v = jnp.mean((x - m) * (x - m), axis=-1, keepdims=True)
    return (x - m) / jnp.sqrt(v + 1e-5) * g + b


def _dot_t(a, b):
    return jax.lax.dot_general(a, b, (((1,), (1,)), ((), ())),
                               preferred_element_type=jnp.float32)


def _qkv_kernel(x_ref, g_ref, b_ref, wqk_ref, wv_ref, qk_ref, v_ref):
    xn = _layernorm(x_ref[...], g_ref[...], b_ref[...])
    qk_ref[...] = _dot_t(xn, wqk_ref[...])
    v_ref[...] = _dot_t(xn, wv_ref[...])


def _qkv(x2, g, b, wqk, wv):
    return pl.pallas_call(
        _qkv_kernel,
        grid=(NROWB,),
        in_specs=[
            pl.BlockSpec((ROWB, EMB), lambda i: (i, 0)),
            pl.BlockSpec((1, EMB), lambda i: (0, 0)),
            pl.BlockSpec((1, EMB), lambda i: (0, 0)),
            pl.BlockSpec((EMB, EMB), lambda i: (0, 0)),
            pl.BlockSpec((EMB, EMB), lambda i: (0, 0)),
        ],
        out_specs=[
            pl.BlockSpec((ROWB, EMB), lambda i: (i, 0)),
            pl.BlockSpec((ROWB, EMB), lambda i: (i, 0)),
        ],
        out_shape=[
            jax.ShapeDtypeStruct((T, EMB), jnp.float32),
            jax.ShapeDtypeStruct((T, EMB), jnp.float32),
        ],
    )(x2, g.reshape(1, EMB), b.reshape(1, EMB), wqk, wv)


def _keys_kernel(qk_ref, rot_ref, key_ref):
    r = pl.program_id(0) % NHASH
    rot = jnp.dot(qk_ref[...], rot_ref[0],
                  preferred_element_type=jnp.float32)
    full = jnp.concatenate([rot, -rot], axis=1)
    mx = jnp.max(full, axis=1, keepdims=True)
    lane = jax.lax.broadcasted_iota(jnp.int32, full.shape, 1)
    am = jnp.min(jnp.where(full == mx, lane, NBUCKETS),
                 axis=1, keepdims=True)
    pos = jax.lax.broadcasted_iota(jnp.int32, (T, 1), 0)
    key_ref[0] = T * am + (T * NBUCKETS) * r + pos


def _sort_keys(qk, rot):
    out = pl.pallas_call(
        _keys_kernel,
        grid=(HEADS * NHASH,),
        in_specs=[
            pl.BlockSpec((T, DH), lambda g: (0, g // NHASH)),
            pl.BlockSpec((1, DH, NBUCKETS // 2), lambda g: (g % NHASH, 0, 0)),
        ],
        out_specs=pl.BlockSpec((1, T, 1), lambda g: (g, 0, 0)),
        out_shape=jax.ShapeDtypeStruct((HEADS * NHASH, T, 1), jnp.int32),
    )(qk, rot)
    return out.reshape(HEADS, NHASH * T)


def _attn_kernel(qc_ref, qp_ref, vc_ref, vp_ref, tq_ref, tkc_ref, tkp_ref,
                 so_ref, sl_ref):
    q = qc_ref[0]
    k = jnp.concatenate([qc_ref[0], qp_ref[0]], axis=0)
    vv = jnp.concatenate([vc_ref[0], vp_ref[0]], axis=0)
    nrm = jnp.sqrt(jnp.sum(k * k, axis=1, keepdims=True))
    kn = k / jnp.maximum(nrm, 1e-6)
    d = _dot_t(q, kn) * (DH ** -0.5)
    tq = tq_ref[0]
    tk = jnp.concatenate([tkc_ref[0], tkp_ref[0]], axis=1)
    d = jnp.where(tq == tk, -5e4, d)
    m = jnp.max(d, axis=1, keepdims=True)
    lse = m + jnp.log(jnp.sum(jnp.exp(d - m), axis=1, keepdims=True))
    p = jnp.exp(d - lse)
    so_ref[0] = jnp.dot(p, vv, preferred_element_type=jnp.float32)
    sl_ref[0] = jnp.broadcast_to(lse, (CS, DH))


def _attention(sqk, sv, st):
    stq = st.reshape(HEADS * NCHUNKS, CS, 1)
    stk = st.reshape(HEADS * NCHUNKS, 1, CS)
    prev = lambda h, c: (h * NCHUNKS + (c + NCHUNKS - 1) % NCHUNKS, 0, 0)
    cur = lambda h, c: (h * NCHUNKS + c, 0, 0)
    return pl.pallas_call(
        _attn_kernel,
        grid=(HEADS, NCHUNKS),
        in_specs=[
            pl.BlockSpec((1, CS, DH), lambda h, c: (h, c, 0)),
            pl.BlockSpec((1, CS, DH), lambda h, c: (h, (c + NCHUNKS - 1) % NCHUNKS, 0)),
            pl.BlockSpec((1, CS, DH), lambda h, c: (h, c, 0)),
            pl.BlockSpec((1, CS, DH), lambda h, c: (h, (c + NCHUNKS - 1) % NCHUNKS, 0)),
            pl.BlockSpec((1, CS, 1), cur),
            pl.BlockSpec((1, 1, CS), cur),
            pl.BlockSpec((1, 1, CS), prev),
        ],
        out_specs=[
            pl.BlockSpec((1, CS, DH), lambda h, c: (h * NCHUNKS + c, 0, 0)),
            pl.BlockSpec((1, CS, DH), lambda h, c: (h * NCHUNKS + c, 0, 0)),
        ],
        out_shape=[
            jax.ShapeDtypeStruct((HEADS * NCHUNKS, CS, DH), jnp.float32),
            jax.ShapeDtypeStruct((HEADS * NCHUNKS, CS, DH), jnp.float32),
        ],
    )(sqk.reshape(HEADS, NHASH * T, DH), sqk.reshape(HEADS, NHASH * T, DH),
      sv.reshape(HEADS, NHASH * T, DH), sv.reshape(HEADS, NHASH * T, DH),
      stq, stk, stk)


def _combine_kernel(o_ref, l_ref, x1_ref, wo_ref, bo_ref, y1_ref):
    l = l_ref[...]
    m = jnp.max(l, axis=1, keepdims=True)
    lse = m + jnp.log(jnp.sum(jnp.exp(l - m), axis=1, keepdims=True))
    p = jnp.exp(l - lse)
    o = jnp.sum(o_ref[...] * p, axis=1)
    y1_ref[...] = x1_ref[...] + _dot_t(o, wo_ref[...]) + bo_ref[...]


def _combine(o_un, l_un, x1, wo, bo):
    return pl.pallas_call(
        _combine_kernel,
        grid=(NROWB,),
        in_specs=[
            pl.BlockSpec((ROWB, NHASH, EMB), lambda i: (i, 0, 0)),
            pl.BlockSpec((ROWB, NHASH, EMB), lambda i: (i, 0, 0)),
            pl.BlockSpec((ROWB, EMB), lambda i: (i, 0)),
            pl.BlockSpec((EMB, EMB), lambda i: (0, 0)),
            pl.BlockSpec((1, EMB), lambda i: (0, 0)),
        ],
        out_specs=pl.BlockSpec((ROWB, EMB), lambda i: (i, 0)),
        out_shape=jax.ShapeDtypeStruct((T, EMB), jnp.float32),
    )(o_un, l_un, x1, wo, bo.reshape(1, EMB))


def _erf(x):
    s = jnp.sign(x)
    a = jnp.abs(x)
    t = 1.0 / (1.0 + 0.3275911 * a)
    y = 1.0 - (((((1.061405429 * t - 1.453152027) * t) + 1.421413741) * t
                - 0.284496736) * t + 0.254829592) * t * jnp.exp(-a * a)
    return s * y


def _ff_kernel(y1_ref, x2_ref, g_ref, b_ref, w1_ref, b1_ref, w2_ref, b2_ref,
               out_ref, *, final):
    j = pl.program_id(1)
    xn = _layernorm(y1_ref[...], g_ref[...], b_ref[...])
    h = _dot_t(xn, w1_ref[...]) + b1_ref[...]
    h = 0.5 * h * (1.0 + _erf(h * (2.0 ** -0.5)))
    part = _dot_t(h, w2_ref[...])

    @pl.when(j == 0)
    def _():
        out_ref[...] = part

    @pl.when(j > 0)
    def _():
        out_ref[...] += part

    @pl.when(j == EMB * 4 // EMB - 1)
    def _():
        extra = x2_ref[...] + b2_ref[...]
        if final:
            extra = extra + y1_ref[...]
        out_ref[...] += extra


def _ff(y1, x2, g, b, w1, b1, w2, b2, final):
    nj = 4
    return pl.pallas_call(
        functools.partial(_ff_kernel, final=final),
        grid=(NROWB, nj),
        in_specs=[
            pl.BlockSpec((ROWB, EMB), lambda i, j: (i, 0)),
            pl.BlockSpec((ROWB, EMB), lambda i, j: (i, 0)),
            pl.BlockSpec((1, EMB), lambda i, j: (0, 0)),
            pl.BlockSpec((1, EMB), lambda i, j: (0, 0)),
            pl.BlockSpec((EMB, EMB), lambda i, j: (j, 0)),
            pl.BlockSpec((1, EMB), lambda i, j: (0, j)),
            pl.BlockSpec((EMB, EMB), lambda i, j: (0, j)),
            pl.BlockSpec((1, EMB), lambda i, j: (0, 0)),
        ],
        out_specs=pl.BlockSpec((ROWB, EMB), lambda i, j: (i, 0)),
        out_shape=jax.ShapeDtypeStruct((T, EMB), jnp.float32),
    )(y1, x2, g.reshape(1, EMB), b.reshape(1, EMB), w1,
      b1.reshape(1, 4 * EMB), w2, b2.reshape(1, EMB))


SC_NW = 32
SC_CH = 128


def _sc_mesh():
    return plsc.VectorSubcoreMesh(core_axis_name="c", subcore_axis_name="s",
                                  num_cores=2, num_subcores=16)


def _sc_gather2(ta, tb, idx, m):
    per_w = m // SC_NW
    nch = per_w // SC_CH
    idx2 = idx.reshape(m // SC_CH, SC_CH)

    @functools.partial(
        pl.kernel,
        out_type=[jax.ShapeDtypeStruct((m, DH), jnp.float32),
                  jax.ShapeDtypeStruct((m, DH), jnp.float32)],
        mesh=_sc_mesh(),
        scratch_types=[
            pltpu.VMEM((nch, SC_CH), jnp.int32),
            pltpu.VMEM((SC_CH, DH), jnp.float32),
            pltpu.VMEM((SC_CH, DH), jnp.float32),
            pltpu.SemaphoreType.DMA,
            pltpu.SemaphoreType.DMA,
        ],
    )
    def k(ta_hbm, tb_hbm, idx_hbm, oa_hbm, ob_hbm, idx_v, ba, bb, gsem, ssem):
        wid = lax.axis_index("s") * 2 + lax.axis_index("c")
        pltpu.sync_copy(idx_hbm.at[pl.ds(wid * nch, nch)], idx_v)

        def body(ch, _):
            row0 = wid * per_w + ch * SC_CH
            ga = pltpu.async_copy(ta_hbm.at[idx_v.at[ch]], ba, gsem)
            gb = pltpu.async_copy(tb_hbm.at[idx_v.at[ch]], bb, gsem)
            ga.wait()
            gb.wait()
            sa = pltpu.async_copy(ba, oa_hbm.at[pl.ds(row0, SC_CH)], ssem)
            sb = pltpu.async_copy(bb, ob_hbm.at[pl.ds(row0, SC_CH)], ssem)
            sa.wait()
            sb.wait()
            return 0

        lax.fori_loop(0, nch, body, 0)

    return k(ta, tb, idx2)


def _sc_scatter2(ra, rb, idx, m):
    per_w = m // SC_NW
    nch = per_w // SC_CH
    idx2 = idx.reshape(m // SC_CH, SC_CH)

    @functools.partial(
        pl.kernel,
        out_type=[jax.ShapeDtypeStruct((m, DH), jnp.float32),
                  jax.ShapeDtypeStruct((m, DH), jnp.float32)],
        mesh=_sc_mesh(),
        scratch_types=[
            pltpu.VMEM((nch, SC_CH), jnp.int32),
            pltpu.VMEM((SC_CH, DH), jnp.float32),
            pltpu.VMEM((SC_CH, DH), jnp.float32),
            pltpu.SemaphoreType.DMA,
            pltpu.SemaphoreType.DMA,
        ],
    )
    def k(ra_hbm, rb_hbm, idx_hbm, oa_hbm, ob_hbm, idx_v, ba, bb, gsem, ssem):
        wid = lax.axis_index("s") * 2 + lax.axis_index("c")
        pltpu.sync_copy(idx_hbm.at[pl.ds(wid * nch, nch)], idx_v)

        def body(ch, _):
            row0 = wid * per_w + ch * SC_CH
            ga = pltpu.async_copy(ra_hbm.at[pl.ds(row0, SC_CH)], ba, gsem)
            gb = pltpu.async_copy(rb_hbm.at[pl.ds(row0, SC_CH)], bb, gsem)
            ga.wait()
            gb.wait()
            sa = pltpu.async_copy(ba, oa_hbm.at[idx_v.at[ch]], ssem)
            sb = pltpu.async_copy(bb, ob_hbm.at[idx_v.at[ch]], ssem)
            sa.wait()
            sb.wait()
            return 0

        lax.fori_loop(0, nch, body, 0)

    return k(ra, rb, idx2)


def _layer(x1, x2, p, rot, final):
    qk, v = _qkv(x2, p['lnf_g'], p['lnf_b'], p['Wqk'], p['Wv'])
    keys = _sort_keys(qk, rot)
    sticker = jnp.argsort(keys, axis=-1).astype(jnp.int32)
    st = sticker % T

    h_ids = jnp.arange(HEADS, dtype=jnp.int32)[:, None]
    gidx = (st * HEADS + h_ids).reshape(-1)
    qk_t = qk.reshape(T * HEADS, DH)
    v_t = v.reshape(T * HEADS, DH)
    m = HEADS * NHASH * T
    sqk, sv = _sc_gather2(qk_t, v_t, gidx, m)
    sqk = sqk.reshape(HEADS, NHASH * T, DH)
    sv = sv.reshape(HEADS, NHASH * T, DH)

    so, sl = _attention(sqk, sv, st)

    r_ids = sticker // T
    dest = (st * (NHASH * HEADS) + r_ids * HEADS + h_ids).reshape(-1)
    o_un, l_un = _sc_scatter2(so.reshape(m, DH), sl.reshape(m, DH), dest, m)
    o_un = o_un.reshape(T, NHASH, EMB)
    l_un = l_un.reshape(T, NHASH, EMB)

    y1 = _combine(o_un, l_un, x1, p['Wo'], p['bo'])
    y2 = _ff(y1, x2, p['lng_g'], p['lng_b'], p['W1'], p['b1'],
             p['W2'], p['b2'], final)
    return y1, y2


def kernel(x, params):
    x0 = x[0]
    x1, x2 = x0, x0
    for i, p in enumerate(params):
        rk = jax.random.fold_in(jax.random.key(42), i)
        rot = jax.random.normal(rk, (DH, NHASH, NBUCKETS // 2), jnp.float32)
        rot = rot.transpose(1, 0, 2)
        final = i == len(params) - 1
        x1, x2 = _layer(x1, x2, p, rot, final)
    return x2[None]


# speedup vs baseline: 1.8828x; 1.8828x over previous
import functools

import jax
import jax.numpy as jnp
from jax import lax
from jax.experimental import pallas as pl
from jax.experimental.pallas import tpu as pltpu
from jax.experimental.pallas import tpu_sc as plsc

EMB = 1024
HEADS = 8
DH = 128
T = 4096
NHASH = 4
NBUCKETS = 64
NCHUNKS = NHASH * NBUCKETS
CS = 64
ROWB = 256
NROWB = T // ROWB


def _layernorm(x, g, b):
    m = jnp.mean(x, axis=-1, keepdims=True)
    v = jnp.mean((x - m) * (x - m), axis=-1, keepdims=True)
    return (x - m) / jnp.sqrt(v + 1e-5) * g + b


def _dot_t(a, b):
    return jax.lax.dot_general(a, b, (((1,), (1,)), ((), ())),
                               preferred_element_type=jnp.float32)


def _qkv_kernel(x_ref, g_ref, b_ref, wqk_ref, wv_ref, qk_ref, v_ref):
    xn = _layernorm(x_ref[...], g_ref[...], b_ref[...])
    qk_ref[...] = _dot_t(xn, wqk_ref[...])
    v_ref[...] = _dot_t(xn, wv_ref[...])


def _qkv(x2, g, b, wqk, wv):
    return pl.pallas_call(
        _qkv_kernel,
        grid=(NROWB,),
        in_specs=[
            pl.BlockSpec((ROWB, EMB), lambda i: (i, 0)),
            pl.BlockSpec((1, EMB), lambda i: (0, 0)),
            pl.BlockSpec((1, EMB), lambda i: (0, 0)),
            pl.BlockSpec((EMB, EMB), lambda i: (0, 0)),
            pl.BlockSpec((EMB, EMB), lambda i: (0, 0)),
        ],
        out_specs=[
            pl.BlockSpec((ROWB, EMB), lambda i: (i, 0)),
            pl.BlockSpec((ROWB, EMB), lambda i: (i, 0)),
        ],
        out_shape=[
            jax.ShapeDtypeStruct((T, EMB), jnp.float32),
            jax.ShapeDtypeStruct((T, EMB), jnp.float32),
        ],
    )(x2, g.reshape(1, EMB), b.reshape(1, EMB), wqk, wv)


def _keys_kernel(qk_ref, rot_ref, key_ref):
    r = pl.program_id(0) % NHASH
    rot = jnp.dot(qk_ref[...], rot_ref[0],
                  preferred_element_type=jnp.float32)
    full = jnp.concatenate([rot, -rot], axis=1)
    mx = jnp.max(full, axis=1, keepdims=True)
    lane = jax.lax.broadcasted_iota(jnp.int32, full.shape, 1)
    am = jnp.min(jnp.where(full == mx, lane, NBUCKETS),
                 axis=1, keepdims=True)
    pos = jax.lax.broadcasted_iota(jnp.int32, (T, 1), 0)
    key_ref[0] = T * am + (T * NBUCKETS) * r + pos


def _sort_keys(qk, rot):
    out = pl.pallas_call(
        _keys_kernel,
        grid=(HEADS * NHASH,),
        in_specs=[
            pl.BlockSpec((T, DH), lambda g: (0, g // NHASH)),
            pl.BlockSpec((1, DH, NBUCKETS // 2), lambda g: (g % NHASH, 0, 0)),
        ],
        out_specs=pl.BlockSpec((1, T, 1), lambda g: (g, 0, 0)),
        out_shape=jax.ShapeDtypeStruct((HEADS * NHASH, T, 1), jnp.int32),
    )(qk, rot)
    return out.reshape(HEADS, NHASH * T)


GRP = 8
GQ = GRP * CS
GK = (GRP + 1) * CS
NGRP = NCHUNKS // GRP


def _attn_kernel(qc_ref, qp_ref, vc_ref, vp_ref, tq_ref, tkc_ref, tkp_ref,
                 so_ref, sl_ref):
    q = qc_ref[0]
    k = jnp.concatenate([qp_ref[0], qc_ref[0]], axis=0)
    vv = jnp.concatenate([vp_ref[0], vc_ref[0]], axis=0)
    nrm = jnp.sqrt(jnp.sum(k * k, axis=1, keepdims=True))
    kn = k / jnp.maximum(nrm, 1e-6)
    d = _dot_t(q, kn) * (DH ** -0.5)
    tq = tq_ref[0]
    tk = jnp.concatenate([tkp_ref[0], tkc_ref[0]], axis=1)
    d = jnp.where(tq == tk, -5e4, d)
    rowg = jax.lax.broadcasted_iota(jnp.int32, (GQ, GK), 0) // CS
    colg = jax.lax.broadcasted_iota(jnp.int32, (GQ, GK), 1) // CS
    dcg = colg - rowg
    d = jnp.where((dcg == 0) | (dcg == 1), d, -1e30)
    m = jnp.max(d, axis=1, keepdims=True)
    lse = m + jnp.log(jnp.sum(jnp.exp(d - m), axis=1, keepdims=True))
    p = jnp.exp(d - lse)
    so_ref[0] = jnp.dot(p, vv, preferred_element_type=jnp.float32)
    sl_ref[0] = jnp.broadcast_to(lse, (GQ, DH))


def _attention(sqk, sv, st):
    stq = st.reshape(HEADS * NGRP, GQ, 1)
    stk = st.reshape(HEADS * NGRP, 1, GQ)
    stkp = st.reshape(HEADS * NCHUNKS, 1, CS)
    pchunk = lambda h, g: (g * GRP + NCHUNKS - 1) % NCHUNKS
    return pl.pallas_call(
        _attn_kernel,
        grid=(HEADS, NGRP),
        in_specs=[
            pl.BlockSpec((1, GQ, DH), lambda h, g: (h, g, 0)),
            pl.BlockSpec((1, CS, DH), lambda h, g: (h, pchunk(h, g), 0)),
            pl.BlockSpec((1, GQ, DH), lambda h, g: (h, g, 0)),
            pl.BlockSpec((1, CS, DH), lambda h, g: (h, pchunk(h, g), 0)),
            pl.BlockSpec((1, GQ, 1), lambda h, g: (h * NGRP + g, 0, 0)),
            pl.BlockSpec((1, 1, GQ), lambda h, g: (h * NGRP + g, 0, 0)),
            pl.BlockSpec((1, 1, CS), lambda h, g: (h * NCHUNKS + pchunk(h, g), 0, 0)),
        ],
        out_specs=[
            pl.BlockSpec((1, GQ, DH), lambda h, g: (h * NGRP + g, 0, 0)),
            pl.BlockSpec((1, GQ, DH), lambda h, g: (h * NGRP + g, 0, 0)),
        ],
        out_shape=[
            jax.ShapeDtypeStruct((HEADS * NGRP, GQ, DH), jnp.float32),
            jax.ShapeDtypeStruct((HEADS * NGRP, GQ, DH), jnp.float32),
        ],
    )(sqk.reshape(HEADS, NHASH * T, DH), sqk.reshape(HEADS, NHASH * T, DH),
      sv.reshape(HEADS, NHASH * T, DH), sv.reshape(HEADS, NHASH * T, DH),
      stq, stk, stkp)


def _combine_kernel(o_ref, l_ref, x1_ref, wo_ref, bo_ref, y1_ref):
    l = l_ref[...]
    m = jnp.max(l, axis=1, keepdims=True)
    lse = m + jnp.log(jnp.sum(jnp.exp(l - m), axis=1, keepdims=True))
    p = jnp.exp(l - lse)
    o = jnp.sum(o_ref[...] * p, axis=1)
    y1_ref[...] = x1_ref[...] + _dot_t(o, wo_ref[...]) + bo_ref[...]


def _combine(o_un, l_un, x1, wo, bo):
    return pl.pallas_call(
        _combine_kernel,
        grid=(NROWB,),
        in_specs=[
            pl.BlockSpec((ROWB, NHASH, EMB), lambda i: (i, 0, 0)),
            pl.BlockSpec((ROWB, NHASH, EMB), lambda i: (i, 0, 0)),
            pl.BlockSpec((ROWB, EMB), lambda i: (i, 0)),
            pl.BlockSpec((EMB, EMB), lambda i: (0, 0)),
            pl.BlockSpec((1, EMB), lambda i: (0, 0)),
        ],
        out_specs=pl.BlockSpec((ROWB, EMB), lambda i: (i, 0)),
        out_shape=jax.ShapeDtypeStruct((T, EMB), jnp.float32),
    )(o_un, l_un, x1, wo, bo.reshape(1, EMB))


def _erf(x):
    s = jnp.sign(x)
    a = jnp.abs(x)
    t = 1.0 / (1.0 + 0.3275911 * a)
    y = 1.0 - (((((1.061405429 * t - 1.453152027) * t) + 1.421413741) * t
                - 0.284496736) * t + 0.254829592) * t * jnp.exp(-a * a)
    return s * y


def _ff_kernel(y1_ref, x2_ref, g_ref, b_ref, w1_ref, b1_ref, w2_ref, b2_ref,
               out_ref, *, final):
    j = pl.program_id(1)
    xn = _layernorm(y1_ref[...], g_ref[...], b_ref[...])
    h = _dot_t(xn, w1_ref[...]) + b1_ref[...]
    h = 0.5 * h * (1.0 + _erf(h * (2.0 ** -0.5)))
    part = _dot_t(h, w2_ref[...])

    @pl.when(j == 0)
    def _():
        out_ref[...] = part

    @pl.when(j > 0)
    def _():
        out_ref[...] += part

    @pl.when(j == EMB * 4 // EMB - 1)
    def _():
        extra = x2_ref[...] + b2_ref[...]
        if final:
            extra = extra + y1_ref[...]
        out_ref[...] += extra


def _ff(y1, x2, g, b, w1, b1, w2, b2, final):
    nj = 4
    return pl.pallas_call(
        functools.partial(_ff_kernel, final=final),
        grid=(NROWB, nj),
        in_specs=[
            pl.BlockSpec((ROWB, EMB), lambda i, j: (i, 0)),
            pl.BlockSpec((ROWB, EMB), lambda i, j: (i, 0)),
            pl.BlockSpec((1, EMB), lambda i, j: (0, 0)),
            pl.BlockSpec((1, EMB), lambda i, j: (0, 0)),
            pl.BlockSpec((EMB, EMB), lambda i, j: (j, 0)),
            pl.BlockSpec((1, EMB), lambda i, j: (0, j)),
            pl.BlockSpec((EMB, EMB), lambda i, j: (0, j)),
            pl.BlockSpec((1, EMB), lambda i, j: (0, 0)),
        ],
        out_specs=pl.BlockSpec((ROWB, EMB), lambda i, j: (i, 0)),
        out_shape=jax.ShapeDtypeStruct((T, EMB), jnp.float32),
    )(y1, x2, g.reshape(1, EMB), b.reshape(1, EMB), w1,
      b1.reshape(1, 4 * EMB), w2, b2.reshape(1, EMB))


SC_NW = 32
SC_CH = 128


def _sc_mesh():
    return plsc.VectorSubcoreMesh(core_axis_name="c", subcore_axis_name="s",
                                  num_cores=2, num_subcores=16)


def _sc_gather2(ta, tb, idx, m):
    per_w = m // SC_NW
    nch = per_w // SC_CH
    idx2 = idx.reshape(m // SC_CH, SC_CH)

    @functools.partial(
        pl.kernel,
        out_type=[jax.ShapeDtypeStruct((m, DH), jnp.float32),
                  jax.ShapeDtypeStruct((m, DH), jnp.float32)],
        mesh=_sc_mesh(),
        scratch_types=[
            pltpu.VMEM((nch, SC_CH), jnp.int32),
            pltpu.VMEM((SC_CH, DH), jnp.float32),
            pltpu.VMEM((SC_CH, DH), jnp.float32),
            pltpu.SemaphoreType.DMA,
            pltpu.SemaphoreType.DMA,
        ],
    )
    def k(ta_hbm, tb_hbm, idx_hbm, oa_hbm, ob_hbm, idx_v, ba, bb, gsem, ssem):
        wid = lax.axis_index("s") * 2 + lax.axis_index("c")
        pltpu.sync_copy(idx_hbm.at[pl.ds(wid * nch, nch)], idx_v)

        def body(ch, _):
            row0 = wid * per_w + ch * SC_CH
            ga = pltpu.async_copy(ta_hbm.at[idx_v.at[ch]], ba, gsem)
            gb = pltpu.async_copy(tb_hbm.at[idx_v.at[ch]], bb, gsem)
            ga.wait()
            gb.wait()
            sa = pltpu.async_copy(ba, oa_hbm.at[pl.ds(row0, SC_CH)], ssem)
            sb = pltpu.async_copy(bb, ob_hbm.at[pl.ds(row0, SC_CH)], ssem)
            sa.wait()
            sb.wait()
            return 0

        lax.fori_loop(0, nch, body, 0)

    return k(ta, tb, idx2)


def _sc_scatter2(ra, rb, idx, m):
    per_w = m // SC_NW
    nch = per_w // SC_CH
    idx2 = idx.reshape(m // SC_CH, SC_CH)

    @functools.partial(
        pl.kernel,
        out_type=[jax.ShapeDtypeStruct((m, DH), jnp.float32),
                  jax.ShapeDtypeStruct((m, DH), jnp.float32)],
        mesh=_sc_mesh(),
        scratch_types=[
            pltpu.VMEM((nch, SC_CH), jnp.int32),
            pltpu.VMEM((SC_CH, DH), jnp.float32),
            pltpu.VMEM((SC_CH, DH), jnp.float32),
            pltpu.SemaphoreType.DMA,
            pltpu.SemaphoreType.DMA,
        ],
    )
    def k(ra_hbm, rb_hbm, idx_hbm, oa_hbm, ob_hbm, idx_v, ba, bb, gsem, ssem):
        wid = lax.axis_index("s") * 2 + lax.axis_index("c")
        pltpu.sync_copy(idx_hbm.at[pl.ds(wid * nch, nch)], idx_v)

        def body(ch, _):
            row0 = wid * per_w + ch * SC_CH
            ga = pltpu.async_copy(ra_hbm.at[pl.ds(row0, SC_CH)], ba, gsem)
            gb = pltpu.async_copy(rb_hbm.at[pl.ds(row0, SC_CH)], bb, gsem)
            ga.wait()
            gb.wait()
            sa = pltpu.async_copy(ba, oa_hbm.at[idx_v.at[ch]], ssem)
            sb = pltpu.async_copy(bb, ob_hbm.at[idx_v.at[ch]], ssem)
            sa.wait()
            sb.wait()
            return 0

        lax.fori_loop(0, nch, body, 0)

    return k(ra, rb, idx2)


def _layer(x1, x2, p, rot, final):
    qk, v = _qkv(x2, p['lnf_g'], p['lnf_b'], p['Wqk'], p['Wv'])
    keys = _sort_keys(qk, rot)
    sticker = jnp.argsort(keys, axis=-1).astype(jnp.int32)
    st = sticker % T

    h_ids = jnp.arange(HEADS, dtype=jnp.int32)[:, None]
    gidx = (st * HEADS + h_ids).reshape(-1)
    qk_t = qk.reshape(T * HEADS, DH)
    v_t = v.reshape(T * HEADS, DH)
    m = HEADS * NHASH * T
    sqk, sv = _sc_gather2(qk_t, v_t, gidx, m)
    sqk = sqk.reshape(HEADS, NHASH * T, DH)
    sv = sv.reshape(HEADS, NHASH * T, DH)

    so, sl = _attention(sqk, sv, st)

    r_ids = sticker // T
    dest = (st * (NHASH * HEADS) + r_ids * HEADS + h_ids).reshape(-1)
    o_un, l_un = _sc_scatter2(so.reshape(m, DH), sl.reshape(m, DH), dest, m)
    o_un = o_un.reshape(T, NHASH, EMB)
    l_un = l_un.reshape(T, NHASH, EMB)

    y1 = _combine(o_un, l_un, x1, p['Wo'], p['bo'])
    y2 = _ff(y1, x2, p['lng_g'], p['lng_b'], p['W1'], p['b1'],
             p['W2'], p['b2'], final)
    return y1, y2


def kernel(x, params):
    x0 = x[0]
    x1, x2 = x0, x0
    for i, p in enumerate(params):
        rk = jax.random.fold_in(jax.random.key(42), i)
        rot = jax.random.normal(rk, (DH, NHASH, NBUCKETS // 2), jnp.float32)
        rot = rot.transpose(1, 0, 2)
        final = i == len(params) - 1
        x1, x2 = _layer(x1, x2, p, rot, final)
    return x2[None]


# speedup vs baseline: 1.9968x; 1.0605x over previous
import functools

import jax
import jax.numpy as jnp
from jax import lax
from jax.experimental import pallas as pl
from jax.experimental.pallas import tpu as pltpu
from jax.experimental.pallas import tpu_sc as plsc

EMB = 1024
HEADS = 8
DH = 128
T = 4096
NHASH = 4
NBUCKETS = 64
NCHUNKS = NHASH * NBUCKETS
CS = 64
ROWB = 512
NROWB = T // ROWB


def _layernorm(x, g, b):
    m = jnp.mean(x, axis=-1, keepdims=True)
    v = jnp.mean((x - m) * (x - m), axis=-1, keepdims=True)
    return (x - m) / jnp.sqrt(v + 1e-5) * g + b


def _dot_t(a, b):
    return jax.lax.dot_general(a, b, (((1,), (1,)), ((), ())),
                               preferred_element_type=jnp.float32)


def _qkv_kernel(x_ref, g_ref, b_ref, wqk_ref, wv_ref, qk_ref, v_ref):
    xn = _layernorm(x_ref[...], g_ref[...], b_ref[...])
    qk_ref[...] = _dot_t(xn, wqk_ref[...])
    v_ref[...] = _dot_t(xn, wv_ref[...])


def _qkv(x2, g, b, wqk, wv):
    return pl.pallas_call(
        _qkv_kernel,
        grid=(NROWB,),
        in_specs=[
            pl.BlockSpec((ROWB, EMB), lambda i: (i, 0)),
            pl.BlockSpec((1, EMB), lambda i: (0, 0)),
            pl.BlockSpec((1, EMB), lambda i: (0, 0)),
            pl.BlockSpec((EMB, EMB), lambda i: (0, 0)),
            pl.BlockSpec((EMB, EMB), lambda i: (0, 0)),
        ],
        out_specs=[
            pl.BlockSpec((ROWB, EMB), lambda i: (i, 0)),
            pl.BlockSpec((ROWB, EMB), lambda i: (i, 0)),
        ],
        out_shape=[
            jax.ShapeDtypeStruct((T, EMB), jnp.float32),
            jax.ShapeDtypeStruct((T, EMB), jnp.float32),
        ],
    )(x2, g.reshape(1, EMB), b.reshape(1, EMB), wqk, wv)


def _keys_kernel(qk_ref, rot_ref, key_ref):
    r = pl.program_id(0) % NHASH
    rot = jnp.dot(qk_ref[...], rot_ref[0],
                  preferred_element_type=jnp.float32)
    full = jnp.concatenate([rot, -rot], axis=1)
    mx = jnp.max(full, axis=1, keepdims=True)
    lane = jax.lax.broadcasted_iota(jnp.int32, full.shape, 1)
    am = jnp.min(jnp.where(full == mx, lane, NBUCKETS),
                 axis=1, keepdims=True)
    pos = jax.lax.broadcasted_iota(jnp.int32, (T, 1), 0)
    key_ref[0] = T * am + (T * NBUCKETS) * r + pos


def _sort_keys(qk, rot):
    out = pl.pallas_call(
        _keys_kernel,
        grid=(HEADS * NHASH,),
        in_specs=[
            pl.BlockSpec((T, DH), lambda g: (0, g // NHASH)),
            pl.BlockSpec((1, DH, NBUCKETS // 2), lambda g: (g % NHASH, 0, 0)),
        ],
        out_specs=pl.BlockSpec((1, T, 1), lambda g: (g, 0, 0)),
        out_shape=jax.ShapeDtypeStruct((HEADS * NHASH, T, 1), jnp.int32),
    )(qk, rot)
    return out.reshape(HEADS, NHASH * T)


GRP = 8
GQ = GRP * CS
GK = (GRP + 1) * CS
NGRP = NCHUNKS // GRP


def _attn_kernel(qc_ref, qp_ref, vc_ref, vp_ref, tq_ref, tkc_ref, tkp_ref,
                 so_ref, sl_ref):
    q = qc_ref[0]
    k = jnp.concatenate([qp_ref[0], qc_ref[0]], axis=0)
    vv = jnp.concatenate([vp_ref[0], vc_ref[0]], axis=0)
    nrm = jnp.sqrt(jnp.sum(k * k, axis=1, keepdims=True))
    kn = k / jnp.maximum(nrm, 1e-6)
    d = _dot_t(q, kn) * (DH ** -0.5)
    tq = tq_ref[0]
    tk = jnp.concatenate([tkp_ref[0], tkc_ref[0]], axis=1)
    d = jnp.where(tq == tk, -5e4, d)
    rowg = jax.lax.broadcasted_iota(jnp.int32, (GQ, GK), 0) // CS
    colg = jax.lax.broadcasted_iota(jnp.int32, (GQ, GK), 1) // CS
    dcg = colg - rowg
    d = jnp.where((dcg == 0) | (dcg == 1), d, -1e30)
    m = jnp.max(d, axis=1, keepdims=True)
    lse = m + jnp.log(jnp.sum(jnp.exp(d - m), axis=1, keepdims=True))
    p = jnp.exp(d - lse)
    so_ref[0] = jnp.dot(p, vv, preferred_element_type=jnp.float32)
    sl_ref[0] = jnp.broadcast_to(lse, (GQ, DH))


def _attention(sqk, sv, st):
    stq = st.reshape(HEADS * NGRP, GQ, 1)
    stk = st.reshape(HEADS * NGRP, 1, GQ)
    stkp = st.reshape(HEADS * NCHUNKS, 1, CS)
    pchunk = lambda h, g: (g * GRP + NCHUNKS - 1) % NCHUNKS
    return pl.pallas_call(
        _attn_kernel,
        grid=(HEADS, NGRP),
        in_specs=[
            pl.BlockSpec((1, GQ, DH), lambda h, g: (h, g, 0)),
            pl.BlockSpec((1, CS, DH), lambda h, g: (h, pchunk(h, g), 0)),
            pl.BlockSpec((1, GQ, DH), lambda h, g: (h, g, 0)),
            pl.BlockSpec((1, CS, DH), lambda h, g: (h, pchunk(h, g), 0)),
            pl.BlockSpec((1, GQ, 1), lambda h, g: (h * NGRP + g, 0, 0)),
            pl.BlockSpec((1, 1, GQ), lambda h, g: (h * NGRP + g, 0, 0)),
            pl.BlockSpec((1, 1, CS), lambda h, g: (h * NCHUNKS + pchunk(h, g), 0, 0)),
        ],
        out_specs=[
            pl.BlockSpec((1, GQ, DH), lambda h, g: (h * NGRP + g, 0, 0)),
            pl.BlockSpec((1, GQ, DH), lambda h, g: (h * NGRP + g, 0, 0)),
        ],
        out_shape=[
            jax.ShapeDtypeStruct((HEADS * NGRP, GQ, DH), jnp.float32),
            jax.ShapeDtypeStruct((HEADS * NGRP, GQ, DH), jnp.float32),
        ],
    )(sqk.reshape(HEADS, NHASH * T, DH), sqk.reshape(HEADS, NHASH * T, DH),
      sv.reshape(HEADS, NHASH * T, DH), sv.reshape(HEADS, NHASH * T, DH),
      stq, stk, stkp)


def _combine_kernel(o_ref, l_ref, x1_ref, wo_ref, bo_ref, y1_ref):
    l = l_ref[...]
    m = jnp.max(l, axis=1, keepdims=True)
    lse = m + jnp.log(jnp.sum(jnp.exp(l - m), axis=1, keepdims=True))
    p = jnp.exp(l - lse)
    o = jnp.sum(o_ref[...] * p, axis=1)
    y1_ref[...] = x1_ref[...] + _dot_t(o, wo_ref[...]) + bo_ref[...]


def _combine(o_un, l_un, x1, wo, bo):
    return pl.pallas_call(
        _combine_kernel,
        grid=(NROWB,),
        in_specs=[
            pl.BlockSpec((ROWB, NHASH, EMB), lambda i: (i, 0, 0)),
            pl.BlockSpec((ROWB, NHASH, EMB), lambda i: (i, 0, 0)),
            pl.BlockSpec((ROWB, EMB), lambda i: (i, 0)),
            pl.BlockSpec((EMB, EMB), lambda i: (0, 0)),
            pl.BlockSpec((1, EMB), lambda i: (0, 0)),
        ],
        out_specs=pl.BlockSpec((ROWB, EMB), lambda i: (i, 0)),
        out_shape=jax.ShapeDtypeStruct((T, EMB), jnp.float32),
    )(o_un, l_un, x1, wo, bo.reshape(1, EMB))


def _erf(x):
    s = jnp.sign(x)
    a = jnp.abs(x)
    t = 1.0 / (1.0 + 0.3275911 * a)
    y = 1.0 - (((((1.061405429 * t - 1.453152027) * t) + 1.421413741) * t
                - 0.284496736) * t + 0.254829592) * t * jnp.exp(-a * a)
    return s * y


def _ff_kernel(y1_ref, x2_ref, g_ref, b_ref, w1_ref, b1_ref, w2_ref, b2_ref,
               out_ref, *, final):
    j = pl.program_id(1)
    xn = _layernorm(y1_ref[...], g_ref[...], b_ref[...])
    h = _dot_t(xn, w1_ref[...]) + b1_ref[...]
    h = 0.5 * h * (1.0 + _erf(h * (2.0 ** -0.5)))
    part = _dot_t(h, w2_ref[...])

    @pl.when(j == 0)
    def _():
        out_ref[...] = part

    @pl.when(j > 0)
    def _():
        out_ref[...] += part

    @pl.when(j == EMB * 4 // EMB - 1)
    def _():
        extra = x2_ref[...] + b2_ref[...]
        if final:
            extra = extra + y1_ref[...]
        out_ref[...] += extra


def _ff(y1, x2, g, b, w1, b1, w2, b2, final):
    nj = 4
    rb = 1024
    return pl.pallas_call(
        functools.partial(_ff_kernel, final=final),
        grid=(T // rb, nj),
        in_specs=[
            pl.BlockSpec((rb, EMB), lambda i, j: (i, 0)),
            pl.BlockSpec((rb, EMB), lambda i, j: (i, 0)),
            pl.BlockSpec((1, EMB), lambda i, j: (0, 0)),
            pl.BlockSpec((1, EMB), lambda i, j: (0, 0)),
            pl.BlockSpec((EMB, EMB), lambda i, j: (j, 0)),
            pl.BlockSpec((1, EMB), lambda i, j: (0, j)),
            pl.BlockSpec((EMB, EMB), lambda i, j: (0, j)),
            pl.BlockSpec((1, EMB), lambda i, j: (0, 0)),
        ],
        out_specs=pl.BlockSpec((rb, EMB), lambda i, j: (i, 0)),
        out_shape=jax.ShapeDtypeStruct((T, EMB), jnp.float32),
    )(y1, x2, g.reshape(1, EMB), b.reshape(1, EMB), w1,
      b1.reshape(1, 4 * EMB), w2, b2.reshape(1, EMB))


SC_NW = 32
SC_CH = 128


def _sc_mesh():
    return plsc.VectorSubcoreMesh(core_axis_name="c", subcore_axis_name="s",
                                  num_cores=2, num_subcores=16)


def _sc_gather2(ta, tb, idx, m):
    per_w = m // SC_NW
    nch = per_w // SC_CH
    idx2 = idx.reshape(m // SC_CH, SC_CH)

    @functools.partial(
        pl.kernel,
        out_type=[jax.ShapeDtypeStruct((m, DH), jnp.float32),
                  jax.ShapeDtypeStruct((m, DH), jnp.float32)],
        mesh=_sc_mesh(),
        scratch_types=[
            pltpu.VMEM((nch, SC_CH), jnp.int32),
            pltpu.VMEM((SC_CH, DH), jnp.float32),
            pltpu.VMEM((SC_CH, DH), jnp.float32),
            pltpu.SemaphoreType.DMA,
            pltpu.SemaphoreType.DMA,
        ],
    )
    def k(ta_hbm, tb_hbm, idx_hbm, oa_hbm, ob_hbm, idx_v, ba, bb, gsem, ssem):
        wid = lax.axis_index("s") * 2 + lax.axis_index("c")
        pltpu.sync_copy(idx_hbm.at[pl.ds(wid * nch, nch)], idx_v)

        def body(ch, _):
            row0 = wid * per_w + ch * SC_CH
            ga = pltpu.async_copy(ta_hbm.at[idx_v.at[ch]], ba, gsem)
            gb = pltpu.async_copy(tb_hbm.at[idx_v.at[ch]], bb, gsem)
            ga.wait()
            gb.wait()
            sa = pltpu.async_copy(ba, oa_hbm.at[pl.ds(row0, SC_CH)], ssem)
            sb = pltpu.async_copy(bb, ob_hbm.at[pl.ds(row0, SC_CH)], ssem)
            sa.wait()
            sb.wait()
            return 0

        lax.fori_loop(0, nch, body, 0)

    return k(ta, tb, idx2)


def _sc_scatter2(ra, rb, idx, m):
    per_w = m // SC_NW
    nch = per_w // SC_CH
    idx2 = idx.reshape(m // SC_CH, SC_CH)

    @functools.partial(
        pl.kernel,
        out_type=[jax.ShapeDtypeStruct((m, DH), jnp.float32),
                  jax.ShapeDtypeStruct((m, DH), jnp.float32)],
        mesh=_sc_mesh(),
        scratch_types=[
            pltpu.VMEM((nch, SC_CH), jnp.int32),
            pltpu.VMEM((SC_CH, DH), jnp.float32),
            pltpu.VMEM((SC_CH, DH), jnp.float32),
            pltpu.SemaphoreType.DMA,
            pltpu.SemaphoreType.DMA,
        ],
    )
    def k(ra_hbm, rb_hbm, idx_hbm, oa_hbm, ob_hbm, idx_v, ba, bb, gsem, ssem):
        wid = lax.axis_index("s") * 2 + lax.axis_index("c")
        pltpu.sync_copy(idx_hbm.at[pl.ds(wid * nch, nch)], idx_v)

        def body(ch, _):
            row0 = wid * per_w + ch * SC_CH
            ga = pltpu.async_copy(ra_hbm.at[pl.ds(row0, SC_CH)], ba, gsem)
            gb = pltpu.async_copy(rb_hbm.at[pl.ds(row0, SC_CH)], bb, gsem)
            ga.wait()
            gb.wait()
            sa = pltpu.async_copy(ba, oa_hbm.at[idx_v.at[ch]], ssem)
            sb = pltpu.async_copy(bb, ob_hbm.at[idx_v.at[ch]], ssem)
            sa.wait()
            sb.wait()
            return 0

        lax.fori_loop(0, nch, body, 0)

    return k(ra, rb, idx2)


def _layer(x1, x2, p, rot, final):
    qk, v = _qkv(x2, p['lnf_g'], p['lnf_b'], p['Wqk'], p['Wv'])
    keys = _sort_keys(qk, rot)
    sticker = jnp.argsort(keys, axis=-1).astype(jnp.int32)
    st = sticker % T

    h_ids = jnp.arange(HEADS, dtype=jnp.int32)[:, None]
    gidx = (st * HEADS + h_ids).reshape(-1)
    qk_t = qk.reshape(T * HEADS, DH)
    v_t = v.reshape(T * HEADS, DH)
    m = HEADS * NHASH * T
    sqk, sv = _sc_gather2(qk_t, v_t, gidx, m)
    sqk = sqk.reshape(HEADS, NHASH * T, DH)
    sv = sv.reshape(HEADS, NHASH * T, DH)

    so, sl = _attention(sqk, sv, st)

    r_ids = sticker // T
    dest = (st * (NHASH * HEADS) + r_ids * HEADS + h_ids).reshape(-1)
    o_un, l_un = _sc_scatter2(so.reshape(m, DH), sl.reshape(m, DH), dest, m)
    o_un = o_un.reshape(T, NHASH, EMB)
    l_un = l_un.reshape(T, NHASH, EMB)

    y1 = _combine(o_un, l_un, x1, p['Wo'], p['bo'])
    y2 = _ff(y1, x2, p['lng_g'], p['lng_b'], p['W1'], p['b1'],
             p['W2'], p['b2'], final)
    return y1, y2


def kernel(x, params):
    x0 = x[0]
    x1, x2 = x0, x0
    for i, p in enumerate(params):
        rk = jax.random.fold_in(jax.random.key(42), i)
        rot = jax.random.normal(rk, (DH, NHASH, NBUCKETS // 2), jnp.float32)
        rot = rot.transpose(1, 0, 2)
        final = i == len(params) - 1
        x1, x2 = _layer(x1, x2, p, rot, final)
    return x2[None]


# speedup vs baseline: 2.0336x; 1.0184x over previous
import functools

import jax
import jax.numpy as jnp
from jax import lax
from jax.experimental import pallas as pl
from jax.experimental.pallas import tpu as pltpu
from jax.experimental.pallas import tpu_sc as plsc

EMB = 1024
HEADS = 8
DH = 128
T = 4096
NHASH = 4
NBUCKETS = 64
NCHUNKS = NHASH * NBUCKETS
CS = 64
ROWB = 512
NROWB = T // ROWB


def _layernorm(x, g, b):
    m = jnp.mean(x, axis=-1, keepdims=True)
    v = jnp.mean((x - m) * (x - m), axis=-1, keepdims=True)
    return (x - m) / jnp.sqrt(v + 1e-5) * g + b


def _dot_t(a, b):
    return jax.lax.dot_general(a, b, (((1,), (1,)), ((), ())),
                               preferred_element_type=jnp.float32)


def _qkv_kernel(x_ref, g_ref, b_ref, wqk_ref, wv_ref, qk_ref, v_ref):
    xn = _layernorm(x_ref[...], g_ref[...], b_ref[...])
    qk_ref[...] = _dot_t(xn, wqk_ref[...])
    v_ref[...] = _dot_t(xn, wv_ref[...])


def _qkv(x2, g, b, wqk, wv):
    return pl.pallas_call(
        _qkv_kernel,
        grid=(NROWB,),
        in_specs=[
            pl.BlockSpec((ROWB, EMB), lambda i: (i, 0)),
            pl.BlockSpec((1, EMB), lambda i: (0, 0)),
            pl.BlockSpec((1, EMB), lambda i: (0, 0)),
            pl.BlockSpec((EMB, EMB), lambda i: (0, 0)),
            pl.BlockSpec((EMB, EMB), lambda i: (0, 0)),
        ],
        out_specs=[
            pl.BlockSpec((ROWB, EMB), lambda i: (i, 0)),
            pl.BlockSpec((ROWB, EMB), lambda i: (i, 0)),
        ],
        out_shape=[
            jax.ShapeDtypeStruct((T, EMB), jnp.float32),
            jax.ShapeDtypeStruct((T, EMB), jnp.float32),
        ],
    )(x2, g.reshape(1, EMB), b.reshape(1, EMB), wqk, wv)


def _keys_kernel(qk_ref, rot_ref, key_ref):
    r = pl.program_id(0) % NHASH
    rot = jnp.dot(qk_ref[...], rot_ref[0],
                  preferred_element_type=jnp.float32)
    full = jnp.concatenate([rot, -rot], axis=1)
    mx = jnp.max(full, axis=1, keepdims=True)
    lane = jax.lax.broadcasted_iota(jnp.int32, full.shape, 1)
    am = jnp.min(jnp.where(full == mx, lane, NBUCKETS),
                 axis=1, keepdims=True)
    pos = jax.lax.broadcasted_iota(jnp.int32, (T, 1), 0)
    key_ref[0] = T * am + (T * NBUCKETS) * r + pos


def _sort_keys(qk, rot):
    out = pl.pallas_call(
        _keys_kernel,
        grid=(HEADS * NHASH,),
        in_specs=[
            pl.BlockSpec((T, DH), lambda g: (0, g // NHASH)),
            pl.BlockSpec((1, DH, NBUCKETS // 2), lambda g: (g % NHASH, 0, 0)),
        ],
        out_specs=pl.BlockSpec((1, T, 1), lambda g: (g, 0, 0)),
        out_shape=jax.ShapeDtypeStruct((HEADS * NHASH, T, 1), jnp.int32),
    )(qk, rot)
    return out.reshape(HEADS, NHASH * T)


LW = 16
GRP = 8
GQ = GRP * CS
GK = (GRP + 1) * CS
NGRP = NCHUNKS // GRP


def _attn_kernel(qc_ref, qp_ref, vc_ref, vp_ref, tq_ref, tkc_ref, tkp_ref,
                 so_ref, sl_ref):
    q = qc_ref[0]
    k = jnp.concatenate([qp_ref[0], qc_ref[0]], axis=0)
    vv = jnp.concatenate([vp_ref[0], vc_ref[0]], axis=0)
    nrm = jnp.sqrt(jnp.sum(k * k, axis=1, keepdims=True))
    kn = k / jnp.maximum(nrm, 1e-6)
    d = _dot_t(q, kn) * (DH ** -0.5)
    tq = tq_ref[0]
    tk = jnp.concatenate([tkp_ref[0], tkc_ref[0]], axis=1)
    d = jnp.where(tq == tk, -5e4, d)
    rowg = jax.lax.broadcasted_iota(jnp.int32, (GQ, GK), 0) // CS
    colg = jax.lax.broadcasted_iota(jnp.int32, (GQ, GK), 1) // CS
    dcg = colg - rowg
    d = jnp.where((dcg == 0) | (dcg == 1), d, -1e30)
    m = jnp.max(d, axis=1, keepdims=True)
    lse = m + jnp.log(jnp.sum(jnp.exp(d - m), axis=1, keepdims=True))
    p = jnp.exp(d - lse)
    so_ref[0] = jnp.dot(p, vv, preferred_element_type=jnp.float32)
    sl_ref[0] = jnp.broadcast_to(lse, (GQ, DH))


def _attention(sqk, sv, st):
    stq = st.reshape(HEADS * NGRP, GQ, 1)
    stk = st.reshape(HEADS * NGRP, 1, GQ)
    stkp = st.reshape(HEADS * NCHUNKS, 1, CS)
    pchunk = lambda h, g: (g * GRP + NCHUNKS - 1) % NCHUNKS
    return pl.pallas_call(
        _attn_kernel,
        grid=(HEADS, NGRP),
        in_specs=[
            pl.BlockSpec((1, GQ, DH), lambda h, g: (h, g, 0)),
            pl.BlockSpec((1, CS, DH), lambda h, g: (h, pchunk(h, g), 0)),
            pl.BlockSpec((1, GQ, DH), lambda h, g: (h, g, 0)),
            pl.BlockSpec((1, CS, DH), lambda h, g: (h, pchunk(h, g), 0)),
            pl.BlockSpec((1, GQ, 1), lambda h, g: (h * NGRP + g, 0, 0)),
            pl.BlockSpec((1, 1, GQ), lambda h, g: (h * NGRP + g, 0, 0)),
            pl.BlockSpec((1, 1, CS), lambda h, g: (h * NCHUNKS + pchunk(h, g), 0, 0)),
        ],
        out_specs=[
            pl.BlockSpec((1, GQ, DH), lambda h, g: (h * NGRP + g, 0, 0)),
            pl.BlockSpec((1, GQ, DH), lambda h, g: (h * NGRP + g, 0, 0)),
        ],
        out_shape=[
            jax.ShapeDtypeStruct((HEADS * NGRP, GQ, DH), jnp.float32),
            jax.ShapeDtypeStruct((HEADS * NGRP, GQ, DH), jnp.float32),
        ],
    )(sqk.reshape(HEADS, NHASH * T, DH), sqk.reshape(HEADS, NHASH * T, DH),
      sv.reshape(HEADS, NHASH * T, DH), sv.reshape(HEADS, NHASH * T, DH),
      stq, stk, stkp)


def _combine_kernel(o_ref, l_ref, x1_ref, wo_ref, bo_ref, y1_ref):
    l = l_ref[...]
    m = jnp.max(l, axis=1, keepdims=True)
    lse = m + jnp.log(jnp.sum(jnp.exp(l - m), axis=1, keepdims=True))
    p = jnp.exp(l - lse)
    o = jnp.sum(o_ref[...] * p, axis=1)
    y1_ref[...] = x1_ref[...] + _dot_t(o, wo_ref[...]) + bo_ref[...]


def _combine(o_un, l_un, x1, wo, bo):
    return pl.pallas_call(
        _combine_kernel,
        grid=(NROWB,),
        in_specs=[
            pl.BlockSpec((ROWB, NHASH, EMB), lambda i: (i, 0, 0)),
            pl.BlockSpec((ROWB, NHASH, EMB), lambda i: (i, 0, 0)),
            pl.BlockSpec((ROWB, EMB), lambda i: (i, 0)),
            pl.BlockSpec((EMB, EMB), lambda i: (0, 0)),
            pl.BlockSpec((1, EMB), lambda i: (0, 0)),
        ],
        out_specs=pl.BlockSpec((ROWB, EMB), lambda i: (i, 0)),
        out_shape=jax.ShapeDtypeStruct((T, EMB), jnp.float32),
    )(o_un, l_un, x1, wo, bo.reshape(1, EMB))


def _erf(x):
    s = jnp.sign(x)
    a = jnp.abs(x)
    t = 1.0 / (1.0 + 0.3275911 * a)
    y = 1.0 - (((((1.061405429 * t - 1.453152027) * t) + 1.421413741) * t
                - 0.284496736) * t + 0.254829592) * t * jnp.exp(-a * a)
    return s * y


def _ff_kernel(y1_ref, x2_ref, g_ref, b_ref, w1_ref, b1_ref, w2_ref, b2_ref,
               out_ref, *, final):
    j = pl.program_id(1)
    xn = _layernorm(y1_ref[...], g_ref[...], b_ref[...])
    h = _dot_t(xn, w1_ref[...]) + b1_ref[...]
    h = 0.5 * h * (1.0 + _erf(h * (2.0 ** -0.5)))
    part = _dot_t(h, w2_ref[...])

    @pl.when(j == 0)
    def _():
        out_ref[...] = part

    @pl.when(j > 0)
    def _():
        out_ref[...] += part

    @pl.when(j == EMB * 4 // EMB - 1)
    def _():
        extra = x2_ref[...] + b2_ref[...]
        if final:
            extra = extra + y1_ref[...]
        out_ref[...] += extra


def _ff(y1, x2, g, b, w1, b1, w2, b2, final):
    nj = 4
    rb = 1024
    return pl.pallas_call(
        functools.partial(_ff_kernel, final=final),
        grid=(T // rb, nj),
        in_specs=[
            pl.BlockSpec((rb, EMB), lambda i, j: (i, 0)),
            pl.BlockSpec((rb, EMB), lambda i, j: (i, 0)),
            pl.BlockSpec((1, EMB), lambda i, j: (0, 0)),
            pl.BlockSpec((1, EMB), lambda i, j: (0, 0)),
            pl.BlockSpec((EMB, EMB), lambda i, j: (j, 0)),
            pl.BlockSpec((1, EMB), lambda i, j: (0, j)),
            pl.BlockSpec((EMB, EMB), lambda i, j: (0, j)),
            pl.BlockSpec((1, EMB), lambda i, j: (0, 0)),
        ],
        out_specs=pl.BlockSpec((rb, EMB), lambda i, j: (i, 0)),
        out_shape=jax.ShapeDtypeStruct((T, EMB), jnp.float32),
    )(y1, x2, g.reshape(1, EMB), b.reshape(1, EMB), w1,
      b1.reshape(1, 4 * EMB), w2, b2.reshape(1, EMB))


SC_NW = 32
SC_CH = 128


def _sc_mesh():
    return plsc.VectorSubcoreMesh(core_axis_name="c", subcore_axis_name="s",
                                  num_cores=2, num_subcores=16)


def _sc_gather2(ta, tb, idx, m):
    per_w = m // SC_NW
    nch = per_w // SC_CH
    idx2 = idx.reshape(m // SC_CH, SC_CH)

    @functools.partial(
        pl.kernel,
        out_type=[jax.ShapeDtypeStruct((m, DH), jnp.float32),
                  jax.ShapeDtypeStruct((m, DH), jnp.float32)],
        mesh=_sc_mesh(),
        scratch_types=[
            pltpu.VMEM((nch, SC_CH), jnp.int32),
            pltpu.VMEM((2, SC_CH, DH), jnp.float32),
            pltpu.VMEM((2, SC_CH, DH), jnp.float32),
            pltpu.SemaphoreType.DMA,
            pltpu.SemaphoreType.DMA,
        ],
    )
    def k(ta_hbm, tb_hbm, idx_hbm, oa_hbm, ob_hbm, idx_v, ba, bb, gsem, ssem):
        wid = lax.axis_index("s") * 2 + lax.axis_index("c")
        pltpu.sync_copy(idx_hbm.at[pl.ds(wid * nch, nch)], idx_v)

        def fire(ch, p):
            pltpu.async_copy(ta_hbm.at[idx_v.at[ch]], ba.at[p], gsem)
            pltpu.async_copy(tb_hbm.at[idx_v.at[ch]], bb.at[p], gsem)

        def wait_store(ch, p):
            row0 = wid * per_w + ch * SC_CH
            pltpu.make_async_copy(ba.at[p], oa_hbm.at[pl.ds(row0, SC_CH)],
                                  ssem).wait()
            pltpu.make_async_copy(bb.at[p], ob_hbm.at[pl.ds(row0, SC_CH)],
                                  ssem).wait()

        fire(0, 0)

        def body(ch, _):
            p = ch % 2

            @pl.when(ch >= 1)
            def _():
                wait_store(ch - 1, 1 - p)

            @pl.when(ch + 1 < nch)
            def _():
                fire(ch + 1, 1 - p)

            pltpu.make_async_copy(ta_hbm.at[idx_v.at[ch]], ba.at[p],
                                  gsem).wait()
            pltpu.make_async_copy(tb_hbm.at[idx_v.at[ch]], bb.at[p],
                                  gsem).wait()
            row0 = wid * per_w + ch * SC_CH
            pltpu.async_copy(ba.at[p], oa_hbm.at[pl.ds(row0, SC_CH)], ssem)
            pltpu.async_copy(bb.at[p], ob_hbm.at[pl.ds(row0, SC_CH)], ssem)
            return 0

        lax.fori_loop(0, nch, body, 0)
        wait_store(nch - 1, (nch - 1) % 2)

    return k(ta, tb, idx2)


def _sc_scatter2(ra, rb, idx, m, wa, wb):
    per_w = m // SC_NW
    nch = per_w // SC_CH
    idx2 = idx.reshape(m // SC_CH, SC_CH)

    @functools.partial(
        pl.kernel,
        out_type=[jax.ShapeDtypeStruct((m, wa), jnp.float32),
                  jax.ShapeDtypeStruct((m, wb), jnp.float32)],
        mesh=_sc_mesh(),
        scratch_types=[
            pltpu.VMEM((nch, SC_CH), jnp.int32),
            pltpu.VMEM((2, SC_CH, wa), jnp.float32),
            pltpu.VMEM((2, SC_CH, wb), jnp.float32),
            pltpu.SemaphoreType.DMA,
            pltpu.SemaphoreType.DMA,
        ],
    )
    def k(ra_hbm, rb_hbm, idx_hbm, oa_hbm, ob_hbm, idx_v, ba, bb, gsem, ssem):
        wid = lax.axis_index("s") * 2 + lax.axis_index("c")
        pltpu.sync_copy(idx_hbm.at[pl.ds(wid * nch, nch)], idx_v)

        def fire(ch, p):
            row0 = wid * per_w + ch * SC_CH
            pltpu.async_copy(ra_hbm.at[pl.ds(row0, SC_CH)], ba.at[p], gsem)
            pltpu.async_copy(rb_hbm.at[pl.ds(row0, SC_CH)], bb.at[p], gsem)

        def wait_store(ch, p):
            pltpu.make_async_copy(ba.at[p], oa_hbm.at[idx_v.at[ch]],
                                  ssem).wait()
            pltpu.make_async_copy(bb.at[p], ob_hbm.at[idx_v.at[ch]],
                                  ssem).wait()

        fire(0, 0)

        def body(ch, _):
            p = ch % 2

            @pl.when(ch >= 1)
            def _():
                wait_store(ch - 1, 1 - p)

            @pl.when(ch + 1 < nch)
            def _():
                fire(ch + 1, 1 - p)

            row0 = wid * per_w + ch * SC_CH
            pltpu.make_async_copy(ra_hbm.at[pl.ds(row0, SC_CH)], ba.at[p],
                                  gsem).wait()
            pltpu.make_async_copy(rb_hbm.at[pl.ds(row0, SC_CH)], bb.at[p],
                                  gsem).wait()
            pltpu.async_copy(ba.at[p], oa_hbm.at[idx_v.at[ch]], ssem)
            pltpu.async_copy(bb.at[p], ob_hbm.at[idx_v.at[ch]], ssem)
            return 0

        lax.fori_loop(0, nch, body, 0)
        wait_store(nch - 1, (nch - 1) % 2)

    return k(ra, rb, idx2)


def _layer(x1, x2, p, rot, final):
    qk, v = _qkv(x2, p['lnf_g'], p['lnf_b'], p['Wqk'], p['Wv'])
    keys = _sort_keys(qk, rot)
    sticker = jnp.argsort(keys, axis=-1).astype(jnp.int32)
    st = sticker % T

    h_ids = jnp.arange(HEADS, dtype=jnp.int32)[:, None]
    gidx = (st * HEADS + h_ids).reshape(-1)
    qk_t = qk.reshape(T * HEADS, DH)
    v_t = v.reshape(T * HEADS, DH)
    m = HEADS * NHASH * T
    sqk, sv = _sc_gather2(qk_t, v_t, gidx, m)
    sqk = sqk.reshape(HEADS, NHASH * T, DH)
    sv = sv.reshape(HEADS, NHASH * T, DH)

    so, sl = _attention(sqk, sv, st)

    r_ids = sticker // T
    dest = (st * (NHASH * HEADS) + r_ids * HEADS + h_ids).reshape(-1)
    o_un, l_un = _sc_scatter2(so.reshape(m, DH), sl.reshape(m, DH), dest, m,
                              DH, DH)
    o_un = o_un.reshape(T, NHASH, EMB)
    l_un = l_un.reshape(T, NHASH, EMB)

    y1 = _combine(o_un, l_un, x1, p['Wo'], p['bo'])
    y2 = _ff(y1, x2, p['lng_g'], p['lng_b'], p['W1'], p['b1'],
             p['W2'], p['b2'], final)
    return y1, y2


def kernel(x, params):
    x0 = x[0]
    x1, x2 = x0, x0
    for i, p in enumerate(params):
        rk = jax.random.fold_in(jax.random.key(42), i)
        rot = jax.random.normal(rk, (DH, NHASH, NBUCKETS // 2), jnp.float32)
        rot = rot.transpose(1, 0, 2)
        final = i == len(params) - 1
        x1, x2 = _layer(x1, x2, p, rot, final)
    return x2[None]


# speedup vs baseline: 2.0398x; 1.0031x over previous
import functools

import jax
import jax.numpy as jnp
from jax import lax
from jax.experimental import pallas as pl
from jax.experimental.pallas import tpu as pltpu
from jax.experimental.pallas import tpu_sc as plsc

EMB = 1024
HEADS = 8
DH = 128
T = 4096
NHASH = 4
NBUCKETS = 64
NCHUNKS = NHASH * NBUCKETS
CS = 64
ROWB = 512
NROWB = T // ROWB


def _layernorm(x, g, b):
    m = jnp.mean(x, axis=-1, keepdims=True)
    v = jnp.mean((x - m) * (x - m), axis=-1, keepdims=True)
    return (x - m) / jnp.sqrt(v + 1e-5) * g + b


def _dot_t(a, b):
    return jax.lax.dot_general(a, b, (((1,), (1,)), ((), ())),
                               preferred_element_type=jnp.float32)


def _qkv_kernel(x_ref, g_ref, b_ref, wqk_ref, wv_ref, qk_ref, v_ref):
    xn = _layernorm(x_ref[...], g_ref[...], b_ref[...])
    qk_ref[...] = _dot_t(xn, wqk_ref[...])
    v_ref[...] = _dot_t(xn, wv_ref[...])


def _qkv(x2, g, b, wqk, wv):
    return pl.pallas_call(
        _qkv_kernel,
        grid=(NROWB,),
        in_specs=[
            pl.BlockSpec((ROWB, EMB), lambda i: (i, 0)),
            pl.BlockSpec((1, EMB), lambda i: (0, 0)),
            pl.BlockSpec((1, EMB), lambda i: (0, 0)),
            pl.BlockSpec((EMB, EMB), lambda i: (0, 0)),
            pl.BlockSpec((EMB, EMB), lambda i: (0, 0)),
        ],
        out_specs=[
            pl.BlockSpec((ROWB, EMB), lambda i: (i, 0)),
            pl.BlockSpec((ROWB, EMB), lambda i: (i, 0)),
        ],
        out_shape=[
            jax.ShapeDtypeStruct((T, EMB), jnp.float32),
            jax.ShapeDtypeStruct((T, EMB), jnp.float32),
        ],
    )(x2, g.reshape(1, EMB), b.reshape(1, EMB), wqk, wv)


def _keys_kernel(qk_ref, rot_ref, key_ref):
    r = pl.program_id(0) % NHASH
    rot = jnp.dot(qk_ref[...], rot_ref[0],
                  preferred_element_type=jnp.float32)
    full = jnp.concatenate([rot, -rot], axis=1)
    mx = jnp.max(full, axis=1, keepdims=True)
    lane = jax.lax.broadcasted_iota(jnp.int32, full.shape, 1)
    am = jnp.min(jnp.where(full == mx, lane, NBUCKETS),
                 axis=1, keepdims=True)
    pos = jax.lax.broadcasted_iota(jnp.int32, (T, 1), 0)
    key_ref[0] = T * am + (T * NBUCKETS) * r + pos


def _sort_keys(qk, rot):
    out = pl.pallas_call(
        _keys_kernel,
        grid=(HEADS * NHASH,),
        in_specs=[
            pl.BlockSpec((T, DH), lambda g: (0, g // NHASH)),
            pl.BlockSpec((1, DH, NBUCKETS // 2), lambda g: (g % NHASH, 0, 0)),
        ],
        out_specs=pl.BlockSpec((1, T, 1), lambda g: (g, 0, 0)),
        out_shape=jax.ShapeDtypeStruct((HEADS * NHASH, T, 1), jnp.int32),
    )(qk, rot)
    return out.reshape(HEADS, NHASH * T)


LW = 16
GRP = 8
GQ = GRP * CS
GK = (GRP + 1) * CS
NGRP = NCHUNKS // GRP


def _attn_kernel(qc_ref, qp_ref, vc_ref, vp_ref, tq_ref, tkc_ref, tkp_ref,
                 so_ref, sl_ref):
    q = qc_ref[0]
    k = jnp.concatenate([qp_ref[0], qc_ref[0]], axis=0)
    vv = jnp.concatenate([vp_ref[0], vc_ref[0]], axis=0)
    nrm = jnp.sqrt(jnp.sum(k * k, axis=1, keepdims=True))
    kn = k / jnp.maximum(nrm, 1e-6)
    d = _dot_t(q, kn) * (DH ** -0.5)
    tq = tq_ref[0]
    tk = jnp.concatenate([tkp_ref[0], tkc_ref[0]], axis=1)
    d = jnp.where(tq == tk, -5e4, d)
    rowg = jax.lax.broadcasted_iota(jnp.int32, (GQ, GK), 0) // CS
    colg = jax.lax.broadcasted_iota(jnp.int32, (GQ, GK), 1) // CS
    dcg = colg - rowg
    d = jnp.where((dcg == 0) | (dcg == 1), d, -1e30)
    m = jnp.max(d, axis=1, keepdims=True)
    lse = m + jnp.log(jnp.sum(jnp.exp(d - m), axis=1, keepdims=True))
    p = jnp.exp(d - lse)
    so_ref[0] = jnp.dot(p, vv, preferred_element_type=jnp.float32)
    sl_ref[0] = jnp.broadcast_to(lse, (GQ, DH))


def _attention(sqk, sv, st):
    stq = st.reshape(HEADS * NGRP, GQ, 1)
    stk = st.reshape(HEADS * NGRP, 1, GQ)
    stkp = st.reshape(HEADS * NCHUNKS, 1, CS)
    pchunk = lambda h, g: (g * GRP + NCHUNKS - 1) % NCHUNKS
    return pl.pallas_call(
        _attn_kernel,
        grid=(HEADS, NGRP),
        in_specs=[
            pl.BlockSpec((1, GQ, DH), lambda h, g: (h, g, 0)),
            pl.BlockSpec((1, CS, DH), lambda h, g: (h, pchunk(h, g), 0)),
            pl.BlockSpec((1, GQ, DH), lambda h, g: (h, g, 0)),
            pl.BlockSpec((1, CS, DH), lambda h, g: (h, pchunk(h, g), 0)),
            pl.BlockSpec((1, GQ, 1), lambda h, g: (h * NGRP + g, 0, 0)),
            pl.BlockSpec((1, 1, GQ), lambda h, g: (h * NGRP + g, 0, 0)),
            pl.BlockSpec((1, 1, CS), lambda h, g: (h * NCHUNKS + pchunk(h, g), 0, 0)),
        ],
        out_specs=[
            pl.BlockSpec((1, GQ, DH), lambda h, g: (h * NGRP + g, 0, 0)),
            pl.BlockSpec((1, GQ, DH), lambda h, g: (h * NGRP + g, 0, 0)),
        ],
        out_shape=[
            jax.ShapeDtypeStruct((HEADS * NGRP, GQ, DH), jnp.float32),
            jax.ShapeDtypeStruct((HEADS * NGRP, GQ, DH), jnp.float32),
        ],
    )(sqk.reshape(HEADS, NHASH * T, DH), sqk.reshape(HEADS, NHASH * T, DH),
      sv.reshape(HEADS, NHASH * T, DH), sv.reshape(HEADS, NHASH * T, DH),
      stq, stk, stkp)


def _combine_kernel(o_ref, l_ref, x1_ref, wo_ref, bo_ref, y1_ref):
    l = l_ref[...]
    m = jnp.max(l, axis=1, keepdims=True)
    lse = m + jnp.log(jnp.sum(jnp.exp(l - m), axis=1, keepdims=True))
    p = jnp.exp(l - lse)
    o = jnp.sum(o_ref[...] * p, axis=1)
    y1_ref[...] = x1_ref[...] + _dot_t(o, wo_ref[...]) + bo_ref[...]


def _combine(o_un, l_un, x1, wo, bo):
    return pl.pallas_call(
        _combine_kernel,
        grid=(NROWB,),
        in_specs=[
            pl.BlockSpec((ROWB, NHASH, EMB), lambda i: (i, 0, 0)),
            pl.BlockSpec((ROWB, NHASH, EMB), lambda i: (i, 0, 0)),
            pl.BlockSpec((ROWB, EMB), lambda i: (i, 0)),
            pl.BlockSpec((EMB, EMB), lambda i: (0, 0)),
            pl.BlockSpec((1, EMB), lambda i: (0, 0)),
        ],
        out_specs=pl.BlockSpec((ROWB, EMB), lambda i: (i, 0)),
        out_shape=jax.ShapeDtypeStruct((T, EMB), jnp.float32),
    )(o_un, l_un, x1, wo, bo.reshape(1, EMB))


def _erf(x):
    s = jnp.sign(x)
    a = jnp.abs(x)
    t = 1.0 / (1.0 + 0.3275911 * a)
    y = 1.0 - (((((1.061405429 * t - 1.453152027) * t) + 1.421413741) * t
                - 0.284496736) * t + 0.254829592) * t * jnp.exp(-a * a)
    return s * y


def _ff_kernel(y1_ref, x2_ref, g_ref, b_ref, w1_ref, b1_ref, w2_ref, b2_ref,
               out_ref, *, final):
    j = pl.program_id(1)
    xn = _layernorm(y1_ref[...], g_ref[...], b_ref[...])
    h = _dot_t(xn, w1_ref[...]) + b1_ref[...]
    h = 0.5 * h * (1.0 + _erf(h * (2.0 ** -0.5)))
    part = _dot_t(h, w2_ref[...])

    @pl.when(j == 0)
    def _():
        out_ref[...] = part

    @pl.when(j > 0)
    def _():
        out_ref[...] += part

    @pl.when(j == EMB * 4 // EMB - 1)
    def _():
        extra = x2_ref[...] + b2_ref[...]
        if final:
            extra = extra + y1_ref[...]
        out_ref[...] += extra


def _ff(y1, x2, g, b, w1, b1, w2, b2, final):
    nj = 4
    rb = 1024
    return pl.pallas_call(
        functools.partial(_ff_kernel, final=final),
        grid=(T // rb, nj),
        in_specs=[
            pl.BlockSpec((rb, EMB), lambda i, j: (i, 0)),
            pl.BlockSpec((rb, EMB), lambda i, j: (i, 0)),
            pl.BlockSpec((1, EMB), lambda i, j: (0, 0)),
            pl.BlockSpec((1, EMB), lambda i, j: (0, 0)),
            pl.BlockSpec((EMB, EMB), lambda i, j: (j, 0)),
            pl.BlockSpec((1, EMB), lambda i, j: (0, j)),
            pl.BlockSpec((EMB, EMB), lambda i, j: (0, j)),
            pl.BlockSpec((1, EMB), lambda i, j: (0, 0)),
        ],
        out_specs=pl.BlockSpec((rb, EMB), lambda i, j: (i, 0)),
        out_shape=jax.ShapeDtypeStruct((T, EMB), jnp.float32),
    )(y1, x2, g.reshape(1, EMB), b.reshape(1, EMB), w1,
      b1.reshape(1, 4 * EMB), w2, b2.reshape(1, EMB))


SC_NW = 32
SC_CH = 128


def _sc_mesh():
    return plsc.VectorSubcoreMesh(core_axis_name="c", subcore_axis_name="s",
                                  num_cores=2, num_subcores=16)


def _sc_gather2(ta, tb, idx, m):
    per_w = m // SC_NW
    nch = per_w // SC_CH
    idx2 = idx.reshape(m // SC_CH, SC_CH)

    @functools.partial(
        pl.kernel,
        out_type=[jax.ShapeDtypeStruct((m, DH), jnp.float32),
                  jax.ShapeDtypeStruct((m, DH), jnp.float32)],
        mesh=_sc_mesh(),
        scratch_types=[
            pltpu.VMEM((nch, SC_CH), jnp.int32),
            pltpu.VMEM((2, SC_CH, DH), jnp.float32),
            pltpu.VMEM((2, SC_CH, DH), jnp.float32),
            pltpu.SemaphoreType.DMA,
            pltpu.SemaphoreType.DMA,
        ],
    )
    def k(ta_hbm, tb_hbm, idx_hbm, oa_hbm, ob_hbm, idx_v, ba, bb, gsem, ssem):
        wid = lax.axis_index("s") * 2 + lax.axis_index("c")
        pltpu.sync_copy(idx_hbm.at[pl.ds(wid * nch, nch)], idx_v)

        def fire(ch, p):
            pltpu.async_copy(ta_hbm.at[idx_v.at[ch]], ba.at[p], gsem)
            pltpu.async_copy(tb_hbm.at[idx_v.at[ch]], bb.at[p], gsem)

        def wait_store(ch, p):
            row0 = wid * per_w + ch * SC_CH
            pltpu.make_async_copy(ba.at[p], oa_hbm.at[pl.ds(row0, SC_CH)],
                                  ssem).wait()
            pltpu.make_async_copy(bb.at[p], ob_hbm.at[pl.ds(row0, SC_CH)],
                                  ssem).wait()

        fire(0, 0)

        def body(ch, _):
            p = ch % 2

            @pl.when(ch >= 1)
            def _():
                wait_store(ch - 1, 1 - p)

            @pl.when(ch + 1 < nch)
            def _():
                fire(ch + 1, 1 - p)

            pltpu.make_async_copy(ta_hbm.at[idx_v.at[ch]], ba.at[p],
                                  gsem).wait()
            pltpu.make_async_copy(tb_hbm.at[idx_v.at[ch]], bb.at[p],
                                  gsem).wait()
            row0 = wid * per_w + ch * SC_CH
            pltpu.async_copy(ba.at[p], oa_hbm.at[pl.ds(row0, SC_CH)], ssem)
            pltpu.async_copy(bb.at[p], ob_hbm.at[pl.ds(row0, SC_CH)], ssem)
            return 0

        lax.fori_loop(0, nch, body, 0)
        wait_store(nch - 1, (nch - 1) % 2)

    return k(ta, tb, idx2)


def _sc_scatter2(ra, rb, idx, m, wa, wb):
    per_w = m // SC_NW
    nch = per_w // SC_CH
    idx2 = idx.reshape(m // SC_CH, SC_CH)

    @functools.partial(
        pl.kernel,
        out_type=[jax.ShapeDtypeStruct((m, wa), jnp.float32),
                  jax.ShapeDtypeStruct((m, wb), jnp.float32)],
        mesh=_sc_mesh(),
        scratch_types=[
            pltpu.VMEM((nch, SC_CH), jnp.int32),
            pltpu.VMEM((2, SC_CH, wa), jnp.float32),
            pltpu.VMEM((2, SC_CH, wb), jnp.float32),
            pltpu.SemaphoreType.DMA,
            pltpu.SemaphoreType.DMA,
        ],
    )
    def k(ra_hbm, rb_hbm, idx_hbm, oa_hbm, ob_hbm, idx_v, ba, bb, gsem, ssem):
        wid = lax.axis_index("s") * 2 + lax.axis_index("c")
        pltpu.sync_copy(idx_hbm.at[pl.ds(wid * nch, nch)], idx_v)

        def fire(ch, p):
            row0 = wid * per_w + ch * SC_CH
            pltpu.async_copy(ra_hbm.at[pl.ds(row0, SC_CH)], ba.at[p], gsem)
            pltpu.async_copy(rb_hbm.at[pl.ds(row0, SC_CH)], bb.at[p], gsem)

        def wait_store(ch, p):
            pltpu.make_async_copy(ba.at[p], oa_hbm.at[idx_v.at[ch]],
                                  ssem).wait()
            pltpu.make_async_copy(bb.at[p], ob_hbm.at[idx_v.at[ch]],
                                  ssem).wait()

        fire(0, 0)

        def body(ch, _):
            p = ch % 2

            @pl.when(ch >= 1)
            def _():
                wait_store(ch - 1, 1 - p)

            @pl.when(ch + 1 < nch)
            def _():
                fire(ch + 1, 1 - p)

            row0 = wid * per_w + ch * SC_CH
            pltpu.make_async_copy(ra_hbm.at[pl.ds(row0, SC_CH)], ba.at[p],
                                  gsem).wait()
            pltpu.make_async_copy(rb_hbm.at[pl.ds(row0, SC_CH)], bb.at[p],
                                  gsem).wait()
            pltpu.async_copy(ba.at[p], oa_hbm.at[idx_v.at[ch]], ssem)
            pltpu.async_copy(bb.at[p], ob_hbm.at[idx_v.at[ch]], ssem)
            return 0

        lax.fori_loop(0, nch, body, 0)
        wait_store(nch - 1, (nch - 1) % 2)

    return k(ra, rb, idx2)


def _layer(x1, x2, p, rot, final):
    qk, v = _qkv(x2, p['lnf_g'], p['lnf_b'], p['Wqk'], p['Wv'])
    keys = _sort_keys(qk, rot)
    skey = jnp.sort(keys, axis=-1)
    st = skey % T
    r_ids = skey // (T * NBUCKETS)

    h_ids = jnp.arange(HEADS, dtype=jnp.int32)[:, None]
    gidx = (st * HEADS + h_ids).reshape(-1)
    qk_t = qk.reshape(T * HEADS, DH)
    v_t = v.reshape(T * HEADS, DH)
    m = HEADS * NHASH * T
    sqk, sv = _sc_gather2(qk_t, v_t, gidx, m)
    sqk = sqk.reshape(HEADS, NHASH * T, DH)
    sv = sv.reshape(HEADS, NHASH * T, DH)

    so, sl = _attention(sqk, sv, st)

    dest = (st * (NHASH * HEADS) + r_ids * HEADS + h_ids).reshape(-1)
    o_un, l_un = _sc_scatter2(so.reshape(m, DH), sl.reshape(m, DH), dest, m,
                              DH, DH)
    o_un = o_un.reshape(T, NHASH, EMB)
    l_un = l_un.reshape(T, NHASH, EMB)

    y1 = _combine(o_un, l_un, x1, p['Wo'], p['bo'])
    y2 = _ff(y1, x2, p['lng_g'], p['lng_b'], p['W1'], p['b1'],
             p['W2'], p['b2'], final)
    return y1, y2


def kernel(x, params):
    x0 = x[0]
    x1, x2 = x0, x0
    for i, p in enumerate(params):
        rk = jax.random.fold_in(jax.random.key(42), i)
        rot = jax.random.normal(rk, (DH, NHASH, NBUCKETS // 2), jnp.float32)
        rot = rot.transpose(1, 0, 2)
        final = i == len(params) - 1
        x1, x2 = _layer(x1, x2, p, rot, final)
    return x2[None]
